# baseline (speedup 1.0000x reference)
import functools

import jax
import jax.numpy as jnp
from jax import lax
from jax.experimental import pallas as pl
from jax.experimental.pallas import tpu as pltpu
from jax.experimental.pallas import tpu_sc as plsc

N = 10000
D = 128
E = 160000

NC = 2
NS = 16
NW = NC * NS
EPW = E // NW
CH = 64
EPW_PAD = 5120
NCHUNK = EPW_PAD // CH
DUMMY = N
NPAD = 10240
RPS = NPAD // NS
ZR = 128

_NEG = -1e9


SBC = 8
NSB = NCHUNK // SBC


def _sc_body(h_hbm, src_hbm, dst_hbm, zeros_hbm,
             agg_out,
             src_v, dst_v, rows_v, rows2_v,
             agg_sh, sem, sem2, sems, sem2s):
  cid = lax.axis_index("c")
  sid = lax.axis_index("s")
  wid = cid * NS + sid

  base = sid * RPS
  for b in range(RPS // ZR):
    pltpu.sync_copy(zeros_hbm, agg_sh.at[pl.ds(base + b * ZR, ZR)])

  pltpu.sync_copy(src_hbm.at[pl.ds(wid * EPW_PAD, EPW_PAD)], src_v)
  pltpu.sync_copy(dst_hbm.at[wid], dst_v)

  plsc.subcore_barrier()

  def _gather_start(j, buf, s):
    return pltpu.async_copy(h_hbm.at[src_v.at[pl.ds(j * CH, CH)]], buf, s)

  def _gather_wait(j, buf, s):
    pltpu.make_async_copy(h_hbm.at[src_v.at[pl.ds(j * CH, CH)]], buf, s).wait()

  def _scat_start(j, buf, s):
    return pltpu.async_copy(buf, agg_sh.at[dst_v.at[j]], s, add=True)

  def _scat_wait(j, buf, s):
    pltpu.make_async_copy(buf, agg_sh.at[dst_v.at[j]], s).wait()

  _gather_start(0, rows_v, sem)

  @pl.loop(0, NCHUNK // 2)
  def _(g):
    j0 = 2 * g

    @pl.when(g > 0)
    def _():
      _scat_wait(j0 - 1, rows2_v, sem2s)

    _gather_wait(j0, rows_v, sem)
    _gather_start(j0 + 1, rows2_v, sem2)
    _scat_start(j0, rows_v, sems)
    _gather_wait(j0 + 1, rows2_v, sem2)
    _scat_wait(j0, rows_v, sems)

    @pl.when(g + 1 < NCHUNK // 2)
    def _():
      _gather_start(j0 + 2, rows_v, sem)

    _scat_start(j0 + 1, rows2_v, sem2s)

  _scat_wait(NCHUNK - 1, rows2_v, sem2s)

  plsc.subcore_barrier()

  for b in range(RPS // ZR):
    off = cid * NPAD + base + b * ZR
    pltpu.sync_copy(agg_sh.at[pl.ds(base + b * ZR, ZR)],
                    agg_out.at[pl.ds(off, ZR)])


@functools.cache
def _get_sc_scatter():
  return pl.kernel(
    _sc_body,
    out_type=jax.ShapeDtypeStruct((NC * NPAD, D), jnp.float32),
    mesh=plsc.VectorSubcoreMesh(core_axis_name="c", subcore_axis_name="s",
                                num_cores=NC, num_subcores=NS),
    scratch_types=[
        pltpu.VMEM((EPW_PAD,), jnp.int32),
        pltpu.VMEM((NCHUNK, CH), jnp.int32),
        pltpu.VMEM((CH, D), jnp.float32),
        pltpu.VMEM((CH, D), jnp.float32),
        pltpu.VMEM_SHARED((NPAD, D), jnp.float32),
        pltpu.SemaphoreType.DMA,
        pltpu.SemaphoreType.DMA,
        pltpu.SemaphoreType.DMA,
        pltpu.SemaphoreType.DMA,
    ],
  )


def _col0(r):
  return lax.broadcasted_iota(jnp.int32, (r, D), 1) == 0


def _proj_rows(y, m0):
  s2 = jnp.sum(jnp.where(m0, 0.0, y * y), axis=1, keepdims=True)
  return jnp.where(m0, jnp.sqrt(1.0 + s2), y)


def _linT(x, w):
  return lax.dot_general(x, w, (((1,), (1,)), ((), ())),
                         precision=lax.Precision.HIGHEST,
                         preferred_element_type=jnp.float32)


def _lorentz_inner(a, b, m0):
  p = a * b
  return jnp.sum(jnp.where(m0, -p, p), axis=1, keepdims=True)


def _lnormalize(c, m0):
  inner = _lorentz_inner(c, c, m0)
  return c / jnp.sqrt(jnp.clip(-inner, 1e-6, None))


def _tc_a_body(x_ref, wagg_ref, x0_ref, h2_ref):
  r = x_ref.shape[0]
  m0 = _col0(r)
  x0 = _proj_rows(x_ref[...], m0)
  h2 = _proj_rows(_linT(x0, wagg_ref[...]), m0)
  x0_ref[...] = x0
  h2_ref[...] = h2


def _tc_b_body(aggp_ref, h_ref, x0_ref, wagg_ref,
               z_ref, mf_ref, h1_ref):
  r = h_ref.shape[0]
  m0 = _col0(r)
  parts = aggp_ref[...]
  agg = parts[0] + parts[1]
  upd = agg[:, 0:1] > 0.5
  nrm = _lnormalize(agg, m0)
  h = h_ref[...]
  z = jnp.where(upd, nrm, h)
  x1 = jnp.where(upd, nrm, x0_ref[...])
  z_ref[...] = z
  mf_ref[...] = jnp.where(jnp.broadcast_to(upd, (r, D)), 1.0, 0.0)
  h1_ref[...] = _proj_rows(_linT(x1, wagg_ref[...]), m0)


def _tc_c_body(aggp_ref, h1_ref, x0_ref, z2_ref, mf_ref,
               wq_ref, wk_ref, wv_ref, out_ref):
  r = h1_ref.shape[0]
  m0 = _col0(r)
  parts = aggp_ref[...]
  agg = parts[0] + parts[1]
  upd1 = agg[:, 0:1] > 0.5
  out1 = jnp.where(upd1, _lnormalize(agg, m0), h1_ref[...])

  x0 = x0_ref[...]
  z2 = z2_ref[...]
  upd2 = mf_ref[...][:, 0:1] > 0.5

  wq = wq_ref[...]
  wk = wk_ref[...]
  wv = wv_ref[...]
  q = _proj_rows(_linT(x0, wq), m0)
  k0 = _proj_rows(_linT(x0, wk), m0)
  k1 = _proj_rows(_linT(z2, wk), m0)
  k2 = _proj_rows(_linT(out1, wk), m0)
  v0 = _proj_rows(_linT(x0, wv), m0)
  v1 = _proj_rows(_linT(z2, wv), m0)
  v2 = _proj_rows(_linT(out1, wv), m0)

  s0 = _lorentz_inner(q, k0, m0)
  s1 = jnp.where(upd2, _lorentz_inner(q, k1, m0), _NEG)
  s2 = jnp.where(upd1, _lorentz_inner(q, k2, m0), _NEG)
  m = jnp.maximum(jnp.maximum(s0, s1), s2)
  e0 = jnp.exp(s0 - m)
  e1 = jnp.exp(s1 - m)
  e2 = jnp.exp(s2 - m)
  c = (e0 * v0 + e1 * v1 + e2 * v2) / (e0 + e1 + e2)
  out_ref[...] = _lnormalize(c, m0)


_R = 1000
_G = N // _R

_rows = lambda i: (i, 0)
_rows3 = lambda i: (0, i, 0)
_whole = lambda i: (0, 0)

_bs_rows = pl.BlockSpec((_R, D), _rows)
_bs_w = pl.BlockSpec((D, D), _whole)
_bs_aggp = pl.BlockSpec((NC, _R, D), _rows3)

_tc_a = pl.pallas_call(
    _tc_a_body,
    grid=(_G,),
    in_specs=[_bs_rows, _bs_w],
    out_specs=[_bs_rows, _bs_rows],
    out_shape=(jax.ShapeDtypeStruct((N, D), jnp.float32),
               jax.ShapeDtypeStruct((N, D), jnp.float32)),
)

_tc_b = pl.pallas_call(
    _tc_b_body,
    grid=(_G,),
    in_specs=[_bs_aggp, _bs_rows, _bs_rows, _bs_w],
    out_specs=[_bs_rows, _bs_rows, _bs_rows],
    out_shape=(jax.ShapeDtypeStruct((N, D), jnp.float32),
               jax.ShapeDtypeStruct((N, D), jnp.float32),
               jax.ShapeDtypeStruct((N, D), jnp.float32)),
)

_tc_c = pl.pallas_call(
    _tc_c_body,
    grid=(_G,),
    in_specs=[_bs_aggp, _bs_rows, _bs_rows, _bs_rows, _bs_rows,
              _bs_w, _bs_w, _bs_w],
    out_specs=_bs_rows,
    out_shape=jax.ShapeDtypeStruct((N, D), jnp.float32),
)


def _prep_edges(edges):
  src = edges[0].astype(jnp.int32).reshape(NW, EPW)
  dst = edges[1].astype(jnp.int32).reshape(NW, EPW)
  pad = EPW_PAD - EPW
  src = jnp.pad(src, ((0, 0), (0, pad)), constant_values=0)
  dst = jnp.pad(dst, ((0, 0), (0, pad)), constant_values=DUMMY)
  return (src.reshape(NW * EPW_PAD), dst.reshape(NW, NCHUNK, CH))


@jax.jit
def kernel(x_H, edge_index_l1, edge_index_l2, Wq, Wk, Wv, W_agg):
  src2, dst2 = _prep_edges(edge_index_l2)
  src1, dst1 = _prep_edges(edge_index_l1)
  zeros = jnp.zeros((ZR, D), jnp.float32)

  sc_scatter = _get_sc_scatter()
  x0, h2 = _tc_a(x_H, W_agg)
  agg2p = sc_scatter(h2, src2, dst2, zeros).reshape(NC, NPAD, D)
  z2, m2f, h1 = _tc_b(agg2p, h2, x0, W_agg)
  agg1p = sc_scatter(h1, src1, dst1, zeros).reshape(NC, NPAD, D)
  return _tc_c(agg1p, h1, x0, z2, m2f, Wq, Wk, Wv)


# speedup vs baseline: 1.0862x; 1.0862x over previous
import functools

import jax
import jax.numpy as jnp
from jax import lax
from jax.experimental import pallas as pl
from jax.experimental.pallas import tpu as pltpu
from jax.experimental.pallas import tpu_sc as plsc

N = 10000
D = 128
E = 160000

NC = 2
NS = 16
NW = NC * NS
EPW = E // NW
CH = 32
EPW_PAD = 5120
NCHUNK = EPW_PAD // CH
DUMMY = N
NPAD = 10240
RPS = NPAD // NS
ZR = 128

_NEG = -1e9


SBC = 8
NSB = NCHUNK // SBC


NBUF = 4
NG = NCHUNK // NBUF


def _sc_body(h_hbm, src_hbm, dst_hbm, zeros_hbm,
             agg_out,
             src_v, dst_v, r0, r1, r2, r3,
             agg_sh, *sems):
  cid = lax.axis_index("c")
  sid = lax.axis_index("s")
  wid = cid * NS + sid
  bufs = (r0, r1, r2, r3)
  gs = sems[:NBUF]
  ss = sems[NBUF:]

  base = sid * RPS
  for b in range(RPS // ZR):
    pltpu.sync_copy(zeros_hbm, agg_sh.at[pl.ds(base + b * ZR, ZR)])

  pltpu.sync_copy(src_hbm.at[pl.ds(wid * EPW_PAD, EPW_PAD)], src_v)
  pltpu.sync_copy(dst_hbm.at[wid], dst_v)

  plsc.subcore_barrier()

  def _gather_start(j, buf, s):
    return pltpu.async_copy(h_hbm.at[src_v.at[pl.ds(j * CH, CH)]], buf, s)

  def _gather_wait(j, buf, s):
    pltpu.make_async_copy(h_hbm.at[src_v.at[pl.ds(j * CH, CH)]], buf, s).wait()

  def _scat_start(j, buf, s):
    return pltpu.async_copy(buf, agg_sh.at[dst_v.at[j]], s, add=True)

  def _scat_wait(j, buf, s):
    pltpu.make_async_copy(buf, agg_sh.at[dst_v.at[j]], s).wait()

  for b in range(NBUF):
    _gather_start(b, bufs[b], gs[b])

  @pl.loop(0, NG)
  def _(g):
    for b in range(NBUF):
      j = NBUF * g + b
      _gather_wait(j, bufs[b], gs[b])
      _scat_start(j, bufs[b], ss[b])

      @pl.when(g + 1 < NG)
      def _():
        _scat_wait(j, bufs[b], ss[b])
        _gather_start(j + NBUF, bufs[b], gs[b])

  for b in range(NBUF):
    _scat_wait(NCHUNK - NBUF + b, bufs[b], ss[b])

  plsc.subcore_barrier()

  for b in range(RPS // ZR):
    off = cid * NPAD + base + b * ZR
    pltpu.sync_copy(agg_sh.at[pl.ds(base + b * ZR, ZR)],
                    agg_out.at[pl.ds(off, ZR)])


@functools.cache
def _get_sc_scatter():
  return pl.kernel(
    _sc_body,
    out_type=jax.ShapeDtypeStruct((NC * NPAD, D), jnp.float32),
    mesh=plsc.VectorSubcoreMesh(core_axis_name="c", subcore_axis_name="s",
                                num_cores=NC, num_subcores=NS),
    scratch_types=[
        pltpu.VMEM((EPW_PAD,), jnp.int32),
        pltpu.VMEM((NCHUNK, CH), jnp.int32),
        pltpu.VMEM((CH, D), jnp.float32),
        pltpu.VMEM((CH, D), jnp.float32),
        pltpu.VMEM((CH, D), jnp.float32),
        pltpu.VMEM((CH, D), jnp.float32),
        pltpu.VMEM_SHARED((NPAD, D), jnp.float32),
    ] + [pltpu.SemaphoreType.DMA] * (2 * NBUF),
  )


def _col0(r):
  return lax.broadcasted_iota(jnp.int32, (r, D), 1) == 0


def _proj_rows(y, m0):
  s2 = jnp.sum(jnp.where(m0, 0.0, y * y), axis=1, keepdims=True)
  return jnp.where(m0, jnp.sqrt(1.0 + s2), y)


def _linT(x, w):
  return lax.dot_general(x, w, (((1,), (1,)), ((), ())),
                         precision=lax.Precision.HIGHEST,
                         preferred_element_type=jnp.float32)


def _lorentz_inner(a, b, m0):
  p = a * b
  return jnp.sum(jnp.where(m0, -p, p), axis=1, keepdims=True)


def _lnormalize(c, m0):
  inner = _lorentz_inner(c, c, m0)
  return c / jnp.sqrt(jnp.clip(-inner, 1e-6, None))


def _tc_a_body(x_ref, wagg_ref, x0_ref, h2_ref):
  r = x_ref.shape[0]
  m0 = _col0(r)
  x0 = _proj_rows(x_ref[...], m0)
  h2 = _proj_rows(_linT(x0, wagg_ref[...]), m0)
  x0_ref[...] = x0
  h2_ref[...] = h2


def _tc_b_body(aggp_ref, h_ref, x0_ref, wagg_ref,
               z_ref, mf_ref, h1_ref):
  r = h_ref.shape[0]
  m0 = _col0(r)
  parts = aggp_ref[...]
  agg = parts[0] + parts[1]
  upd = agg[:, 0:1] > 0.5
  nrm = _lnormalize(agg, m0)
  h = h_ref[...]
  z = jnp.where(upd, nrm, h)
  x1 = jnp.where(upd, nrm, x0_ref[...])
  z_ref[...] = z
  mf_ref[...] = jnp.where(jnp.broadcast_to(upd, (r, D)), 1.0, 0.0)
  h1_ref[...] = _proj_rows(_linT(x1, wagg_ref[...]), m0)


def _tc_c_body(aggp_ref, h1_ref, x0_ref, z2_ref, mf_ref,
               wq_ref, wk_ref, wv_ref, out_ref):
  r = h1_ref.shape[0]
  m0 = _col0(r)
  parts = aggp_ref[...]
  agg = parts[0] + parts[1]
  upd1 = agg[:, 0:1] > 0.5
  out1 = jnp.where(upd1, _lnormalize(agg, m0), h1_ref[...])

  x0 = x0_ref[...]
  z2 = z2_ref[...]
  upd2 = mf_ref[...][:, 0:1] > 0.5

  wq = wq_ref[...]
  wk = wk_ref[...]
  wv = wv_ref[...]
  q = _proj_rows(_linT(x0, wq), m0)
  k0 = _proj_rows(_linT(x0, wk), m0)
  k1 = _proj_rows(_linT(z2, wk), m0)
  k2 = _proj_rows(_linT(out1, wk), m0)
  v0 = _proj_rows(_linT(x0, wv), m0)
  v1 = _proj_rows(_linT(z2, wv), m0)
  v2 = _proj_rows(_linT(out1, wv), m0)

  s0 = _lorentz_inner(q, k0, m0)
  s1 = jnp.where(upd2, _lorentz_inner(q, k1, m0), _NEG)
  s2 = jnp.where(upd1, _lorentz_inner(q, k2, m0), _NEG)
  m = jnp.maximum(jnp.maximum(s0, s1), s2)
  e0 = jnp.exp(s0 - m)
  e1 = jnp.exp(s1 - m)
  e2 = jnp.exp(s2 - m)
  c = (e0 * v0 + e1 * v1 + e2 * v2) / (e0 + e1 + e2)
  out_ref[...] = _lnormalize(c, m0)


_R = 1000
_G = N // _R

_rows = lambda i: (i, 0)
_rows3 = lambda i: (0, i, 0)
_whole = lambda i: (0, 0)

_bs_rows = pl.BlockSpec((_R, D), _rows)
_bs_w = pl.BlockSpec((D, D), _whole)
_bs_aggp = pl.BlockSpec((NC, _R, D), _rows3)

_tc_a = pl.pallas_call(
    _tc_a_body,
    grid=(_G,),
    in_specs=[_bs_rows, _bs_w],
    out_specs=[_bs_rows, _bs_rows],
    out_shape=(jax.ShapeDtypeStruct((N, D), jnp.float32),
               jax.ShapeDtypeStruct((N, D), jnp.float32)),
)

_tc_b = pl.pallas_call(
    _tc_b_body,
    grid=(_G,),
    in_specs=[_bs_aggp, _bs_rows, _bs_rows, _bs_w],
    out_specs=[_bs_rows, _bs_rows, _bs_rows],
    out_shape=(jax.ShapeDtypeStruct((N, D), jnp.float32),
               jax.ShapeDtypeStruct((N, D), jnp.float32),
               jax.ShapeDtypeStruct((N, D), jnp.float32)),
)

_tc_c = pl.pallas_call(
    _tc_c_body,
    grid=(_G,),
    in_specs=[_bs_aggp, _bs_rows, _bs_rows, _bs_rows, _bs_rows,
              _bs_w, _bs_w, _bs_w],
    out_specs=_bs_rows,
    out_shape=jax.ShapeDtypeStruct((N, D), jnp.float32),
)


def _prep_edges(edges):
  src = edges[0].astype(jnp.int32).reshape(NW, EPW)
  dst = edges[1].astype(jnp.int32).reshape(NW, EPW)
  pad = EPW_PAD - EPW
  src = jnp.pad(src, ((0, 0), (0, pad)), constant_values=0)
  dst = jnp.pad(dst, ((0, 0), (0, pad)), constant_values=DUMMY)
  return (src.reshape(NW * EPW_PAD), dst.reshape(NW, NCHUNK, CH))


@jax.jit
def kernel(x_H, edge_index_l1, edge_index_l2, Wq, Wk, Wv, W_agg):
  src2, dst2 = _prep_edges(edge_index_l2)
  src1, dst1 = _prep_edges(edge_index_l1)
  zeros = jnp.zeros((ZR, D), jnp.float32)

  sc_scatter = _get_sc_scatter()
  x0, h2 = _tc_a(x_H, W_agg)
  agg2p = sc_scatter(h2, src2, dst2, zeros).reshape(NC, NPAD, D)
  z2, m2f, h1 = _tc_b(agg2p, h2, x0, W_agg)
  agg1p = sc_scatter(h1, src1, dst1, zeros).reshape(NC, NPAD, D)
  return _tc_c(agg1p, h1, x0, z2, m2f, Wq, Wk, Wv)


# speedup vs baseline: 1.0967x; 1.0097x over previous
import functools

import jax
import jax.numpy as jnp
from jax import lax
from jax.experimental import pallas as pl
from jax.experimental.pallas import tpu as pltpu
from jax.experimental.pallas import tpu_sc as plsc

N = 10000
D = 128
E = 160000

NC = 2
NS = 16
NW = NC * NS
EPW = E // NW
CH = 32
EPW_PAD = 5120
NCHUNK = EPW_PAD // CH
DUMMY = N
NPAD = 10240
RPS = NPAD // NS
ZR = 128

_NEG = -1e9


SBC = 8
NSB = NCHUNK // SBC


NBUF = 4
NG = NCHUNK // NBUF


def _sc_body(h_hbm, src_hbm, dst_hbm, zeros_hbm,
             agg_out,
             src_v, dst_v, r0, r1, r2, r3,
             agg_sh, *sems):
  cid = lax.axis_index("c")
  sid = lax.axis_index("s")
  wid = cid * NS + sid
  bufs = (r0, r1, r2, r3)
  gs = sems[:NBUF]
  ss = sems[NBUF:]

  pltpu.sync_copy(src_hbm.at[pl.ds(wid * EPW_PAD, EPW_PAD)], src_v)
  pltpu.sync_copy(dst_hbm.at[wid], dst_v)

  def _gather_start(j, buf, s):
    return pltpu.async_copy(h_hbm.at[src_v.at[pl.ds(j * CH, CH)]], buf, s)

  def _gather_wait(j, buf, s):
    pltpu.make_async_copy(h_hbm.at[src_v.at[pl.ds(j * CH, CH)]], buf, s).wait()

  def _scat_start(j, buf, s):
    return pltpu.async_copy(buf, agg_sh.at[dst_v.at[j]], s, add=True)

  def _scat_wait(j, buf, s):
    pltpu.make_async_copy(buf, agg_sh.at[dst_v.at[j]], s).wait()

  for b in range(NBUF):
    _gather_start(b, bufs[b], gs[b])

  base = sid * RPS
  for b in range(RPS // ZR):
    pltpu.sync_copy(zeros_hbm, agg_sh.at[pl.ds(base + b * ZR, ZR)])

  plsc.subcore_barrier()

  @pl.loop(0, NG)
  def _(g):
    for b in range(NBUF):
      j = NBUF * g + b
      _gather_wait(j, bufs[b], gs[b])
      _scat_start(j, bufs[b], ss[b])

      @pl.when(g + 1 < NG)
      def _():
        _scat_wait(j, bufs[b], ss[b])
        _gather_start(j + NBUF, bufs[b], gs[b])

  for b in range(NBUF):
    _scat_wait(NCHUNK - NBUF + b, bufs[b], ss[b])

  plsc.subcore_barrier()

  for b in range(RPS // ZR):
    off = cid * NPAD + base + b * ZR
    pltpu.sync_copy(agg_sh.at[pl.ds(base + b * ZR, ZR)],
                    agg_out.at[pl.ds(off, ZR)])


@functools.cache
def _get_sc_scatter():
  return pl.kernel(
    _sc_body,
    out_type=jax.ShapeDtypeStruct((NC * NPAD, D), jnp.float32),
    mesh=plsc.VectorSubcoreMesh(core_axis_name="c", subcore_axis_name="s",
                                num_cores=NC, num_subcores=NS),
    scratch_types=[
        pltpu.VMEM((EPW_PAD,), jnp.int32),
        pltpu.VMEM((NCHUNK, CH), jnp.int32),
    ] + [pltpu.VMEM((CH, D), jnp.float32)] * NBUF + [
        pltpu.VMEM_SHARED((NPAD, D), jnp.float32),
    ] + [pltpu.SemaphoreType.DMA] * (2 * NBUF),
  )


def _col0(r):
  return lax.broadcasted_iota(jnp.int32, (r, D), 1) == 0


def _proj_rows(y, m0):
  s2 = jnp.sum(jnp.where(m0, 0.0, y * y), axis=1, keepdims=True)
  return jnp.where(m0, jnp.sqrt(1.0 + s2), y)


def _linT(x, w):
  return lax.dot_general(x, w, (((1,), (1,)), ((), ())),
                         precision=lax.Precision.HIGHEST,
                         preferred_element_type=jnp.float32)


def _lorentz_inner(a, b, m0):
  p = a * b
  return jnp.sum(jnp.where(m0, -p, p), axis=1, keepdims=True)


def _lnormalize(c, m0):
  inner = _lorentz_inner(c, c, m0)
  return c / jnp.sqrt(jnp.clip(-inner, 1e-6, None))


def _tc_a_body(x_ref, wagg_ref, x0_ref, h2_ref):
  r = x_ref.shape[0]
  m0 = _col0(r)
  x0 = _proj_rows(x_ref[...], m0)
  h2 = _proj_rows(_linT(x0, wagg_ref[...]), m0)
  x0_ref[...] = x0
  h2_ref[...] = h2


def _tc_b_body(aggp_ref, h_ref, x0_ref, wagg_ref,
               z_ref, mf_ref, h1_ref):
  r = h_ref.shape[0]
  m0 = _col0(r)
  parts = aggp_ref[...]
  agg = parts[0] + parts[1]
  upd = agg[:, 0:1] > 0.5
  nrm = _lnormalize(agg, m0)
  h = h_ref[...]
  z = jnp.where(upd, nrm, h)
  x1 = jnp.where(upd, nrm, x0_ref[...])
  z_ref[...] = z
  mf_ref[...] = jnp.where(jnp.broadcast_to(upd, (r, D)), 1.0, 0.0)
  h1_ref[...] = _proj_rows(_linT(x1, wagg_ref[...]), m0)


def _tc_qkv_body(x_ref, wq_ref, wk_ref, wv_ref, q_ref, k_ref, v_ref):
  r = x_ref.shape[0]
  m0 = _col0(r)
  x = x_ref[...]
  q_ref[...] = _proj_rows(_linT(x, wq_ref[...]), m0)
  k_ref[...] = _proj_rows(_linT(x, wk_ref[...]), m0)
  v_ref[...] = _proj_rows(_linT(x, wv_ref[...]), m0)


def _tc_kv_body(x_ref, wk_ref, wv_ref, k_ref, v_ref):
  r = x_ref.shape[0]
  m0 = _col0(r)
  x = x_ref[...]
  k_ref[...] = _proj_rows(_linT(x, wk_ref[...]), m0)
  v_ref[...] = _proj_rows(_linT(x, wv_ref[...]), m0)


def _tc_c_body(aggp_ref, h1_ref, mf_ref, q_ref, k0_ref, v0_ref,
               k1_ref, v1_ref, wk_ref, wv_ref, out_ref):
  r = h1_ref.shape[0]
  m0 = _col0(r)
  parts = aggp_ref[...]
  agg = parts[0] + parts[1]
  upd1 = agg[:, 0:1] > 0.5
  out1 = jnp.where(upd1, _lnormalize(agg, m0), h1_ref[...])

  upd2 = mf_ref[...][:, 0:1] > 0.5

  q = q_ref[...]
  k0 = k0_ref[...]
  k1 = k1_ref[...]
  k2 = _proj_rows(_linT(out1, wk_ref[...]), m0)
  v0 = v0_ref[...]
  v1 = v1_ref[...]
  v2 = _proj_rows(_linT(out1, wv_ref[...]), m0)

  s0 = _lorentz_inner(q, k0, m0)
  s1 = jnp.where(upd2, _lorentz_inner(q, k1, m0), _NEG)
  s2 = jnp.where(upd1, _lorentz_inner(q, k2, m0), _NEG)
  m = jnp.maximum(jnp.maximum(s0, s1), s2)
  e0 = jnp.exp(s0 - m)
  e1 = jnp.exp(s1 - m)
  e2 = jnp.exp(s2 - m)
  c = (e0 * v0 + e1 * v1 + e2 * v2) / (e0 + e1 + e2)
  out_ref[...] = _lnormalize(c, m0)


_R = 1000
_G = N // _R

_rows = lambda i: (i, 0)
_rows3 = lambda i: (0, i, 0)
_whole = lambda i: (0, 0)

_bs_rows = pl.BlockSpec((_R, D), _rows)
_bs_w = pl.BlockSpec((D, D), _whole)
_bs_aggp = pl.BlockSpec((NC, _R, D), _rows3)

_tc_a = pl.pallas_call(
    _tc_a_body,
    grid=(_G,),
    in_specs=[_bs_rows, _bs_w],
    out_specs=[_bs_rows, _bs_rows],
    out_shape=(jax.ShapeDtypeStruct((N, D), jnp.float32),
               jax.ShapeDtypeStruct((N, D), jnp.float32)),
)

_tc_b = pl.pallas_call(
    _tc_b_body,
    grid=(_G,),
    in_specs=[_bs_aggp, _bs_rows, _bs_rows, _bs_w],
    out_specs=[_bs_rows, _bs_rows, _bs_rows],
    out_shape=(jax.ShapeDtypeStruct((N, D), jnp.float32),
               jax.ShapeDtypeStruct((N, D), jnp.float32),
               jax.ShapeDtypeStruct((N, D), jnp.float32)),
)

_tc_qkv = pl.pallas_call(
    _tc_qkv_body,
    grid=(_G,),
    in_specs=[_bs_rows, _bs_w, _bs_w, _bs_w],
    out_specs=[_bs_rows, _bs_rows, _bs_rows],
    out_shape=(jax.ShapeDtypeStruct((N, D), jnp.float32),) * 3,
)

_tc_kv = pl.pallas_call(
    _tc_kv_body,
    grid=(_G,),
    in_specs=[_bs_rows, _bs_w, _bs_w],
    out_specs=[_bs_rows, _bs_rows],
    out_shape=(jax.ShapeDtypeStruct((N, D), jnp.float32),) * 2,
)

_tc_c = pl.pallas_call(
    _tc_c_body,
    grid=(_G,),
    in_specs=[_bs_aggp] + [_bs_rows] * 7 + [_bs_w, _bs_w],
    out_specs=_bs_rows,
    out_shape=jax.ShapeDtypeStruct((N, D), jnp.float32),
)


def _prep_edges(edges):
  src = edges[0].astype(jnp.int32).reshape(NW, EPW)
  dst = edges[1].astype(jnp.int32).reshape(NW, EPW)
  pad = EPW_PAD - EPW
  src = jnp.pad(src, ((0, 0), (0, pad)), constant_values=0)
  dst = jnp.pad(dst, ((0, 0), (0, pad)), constant_values=DUMMY)
  return (src.reshape(NW * EPW_PAD), dst.reshape(NW, NCHUNK, CH))


@jax.jit
def kernel(x_H, edge_index_l1, edge_index_l2, Wq, Wk, Wv, W_agg):
  src2, dst2 = _prep_edges(edge_index_l2)
  src1, dst1 = _prep_edges(edge_index_l1)
  zeros = jnp.zeros((ZR, D), jnp.float32)

  sc_scatter = _get_sc_scatter()
  x0, h2 = _tc_a(x_H, W_agg)
  agg2p = sc_scatter(h2, src2, dst2, zeros).reshape(NC, NPAD, D)
  q, k0, v0 = _tc_qkv(x0, Wq, Wk, Wv)
  z2, m2f, h1 = _tc_b(agg2p, h2, x0, W_agg)
  agg1p = sc_scatter(h1, src1, dst1, zeros).reshape(NC, NPAD, D)
  k1, v1 = _tc_kv(z2, Wk, Wv)
  return _tc_c(agg1p, h1, m2f, q, k0, v0, k1, v1, Wk, Wv)


# speedup vs baseline: 1.1178x; 1.0192x over previous
import functools

import jax
import jax.numpy as jnp
from jax import lax
from jax.experimental import pallas as pl
from jax.experimental.pallas import tpu as pltpu
from jax.experimental.pallas import tpu_sc as plsc

N = 10000
D = 128
E = 160000

NC = 2
NS = 16
NW = NC * NS
EPW = E // NW
CH = 32
EPW_PAD = 5120
NCHUNK = EPW_PAD // CH
DUMMY = N
NPAD = 10240
RPS = NPAD // NS
ZR = 128

_NEG = -1e9


SBC = 8
NSB = NCHUNK // SBC


NBUF = 5
NG = NCHUNK // NBUF


def _sc_body(h_hbm, src_hbm, dst_hbm, zeros_hbm,
             agg_out,
             src_v, dst_v, r0, r1, r2, r3, r4,
             agg_sh, *sems):
  cid = lax.axis_index("c")
  sid = lax.axis_index("s")
  wid = cid * NS + sid
  bufs = (r0, r1, r2, r3, r4)
  gs = sems[:NBUF]
  ss = sems[NBUF:]

  pltpu.sync_copy(src_hbm.at[pl.ds(wid * EPW_PAD, EPW_PAD)], src_v)
  pltpu.sync_copy(dst_hbm.at[wid], dst_v)

  def _gather_start(j, buf, s):
    return pltpu.async_copy(h_hbm.at[src_v.at[pl.ds(j * CH, CH)]], buf, s)

  def _gather_wait(j, buf, s):
    pltpu.make_async_copy(h_hbm.at[src_v.at[pl.ds(j * CH, CH)]], buf, s).wait()

  def _scat_start(j, buf, s):
    return pltpu.async_copy(buf, agg_sh.at[dst_v.at[j]], s, add=True)

  def _scat_wait(j, buf, s):
    pltpu.make_async_copy(buf, agg_sh.at[dst_v.at[j]], s).wait()

  for b in range(NBUF):
    _gather_start(b, bufs[b], gs[b])

  base = sid * RPS
  for b in range(RPS // ZR):
    pltpu.sync_copy(zeros_hbm, agg_sh.at[pl.ds(base + b * ZR, ZR)])

  plsc.subcore_barrier()

  @pl.loop(0, NG)
  def _(g):
    for b in range(NBUF):
      j = NBUF * g + b
      _gather_wait(j, bufs[b], gs[b])
      _scat_start(j, bufs[b], ss[b])

      @pl.when(g + 1 < NG)
      def _():
        _scat_wait(j, bufs[b], ss[b])
        _gather_start(j + NBUF, bufs[b], gs[b])

  for b in range(NBUF):
    _scat_wait(NCHUNK - NBUF + b, bufs[b], ss[b])

  plsc.subcore_barrier()

  for b in range(RPS // ZR):
    off = cid * NPAD + base + b * ZR
    pltpu.sync_copy(agg_sh.at[pl.ds(base + b * ZR, ZR)],
                    agg_out.at[pl.ds(off, ZR)])


@functools.cache
def _get_sc_scatter():
  return pl.kernel(
    _sc_body,
    out_type=jax.ShapeDtypeStruct((NC * NPAD, D), jnp.float32),
    mesh=plsc.VectorSubcoreMesh(core_axis_name="c", subcore_axis_name="s",
                                num_cores=NC, num_subcores=NS),
    scratch_types=[
        pltpu.VMEM((EPW_PAD,), jnp.int32),
        pltpu.VMEM((NCHUNK, CH), jnp.int32),
    ] + [pltpu.VMEM((CH, D), jnp.float32)] * NBUF + [
        pltpu.VMEM_SHARED((NPAD, D), jnp.float32),
    ] + [pltpu.SemaphoreType.DMA] * (2 * NBUF),
  )


def _col0(r):
  return lax.broadcasted_iota(jnp.int32, (r, D), 1) == 0


def _proj_rows(y, m0):
  s2 = jnp.sum(jnp.where(m0, 0.0, y * y), axis=1, keepdims=True)
  return jnp.where(m0, jnp.sqrt(1.0 + s2), y)


def _linT(x, w):
  return lax.dot_general(x, w, (((1,), (1,)), ((), ())),
                         precision=lax.Precision.HIGHEST,
                         preferred_element_type=jnp.float32)


def _lorentz_inner(a, b, m0):
  p = a * b
  return jnp.sum(jnp.where(m0, -p, p), axis=1, keepdims=True)


def _lnormalize(c, m0):
  inner = _lorentz_inner(c, c, m0)
  return c / jnp.sqrt(jnp.clip(-inner, 1e-6, None))


def _tc_a_body(x_ref, wagg_ref, x0_ref, h2_ref):
  r = x_ref.shape[0]
  m0 = _col0(r)
  x0 = _proj_rows(x_ref[...], m0)
  h2 = _proj_rows(_linT(x0, wagg_ref[...]), m0)
  x0_ref[...] = x0
  h2_ref[...] = h2


def _tc_b_body(aggp_ref, h_ref, x0_ref, wagg_ref,
               z_ref, mf_ref, h1_ref):
  r = h_ref.shape[0]
  m0 = _col0(r)
  parts = aggp_ref[...]
  agg = parts[0] + parts[1]
  upd = agg[:, 0:1] > 0.5
  nrm = _lnormalize(agg, m0)
  h = h_ref[...]
  z = jnp.where(upd, nrm, h)
  x1 = jnp.where(upd, nrm, x0_ref[...])
  z_ref[...] = z
  mf_ref[...] = jnp.where(jnp.broadcast_to(upd, (r, D)), 1.0, 0.0)
  h1_ref[...] = _proj_rows(_linT(x1, wagg_ref[...]), m0)


def _tc_qkv_body(x_ref, wq_ref, wk_ref, wv_ref, q_ref, k_ref, v_ref):
  r = x_ref.shape[0]
  m0 = _col0(r)
  x = x_ref[...]
  q_ref[...] = _proj_rows(_linT(x, wq_ref[...]), m0)
  k_ref[...] = _proj_rows(_linT(x, wk_ref[...]), m0)
  v_ref[...] = _proj_rows(_linT(x, wv_ref[...]), m0)


def _tc_kv_body(x_ref, wk_ref, wv_ref, k_ref, v_ref):
  r = x_ref.shape[0]
  m0 = _col0(r)
  x = x_ref[...]
  k_ref[...] = _proj_rows(_linT(x, wk_ref[...]), m0)
  v_ref[...] = _proj_rows(_linT(x, wv_ref[...]), m0)


def _tc_c_body(aggp_ref, h1_ref, mf_ref, q_ref, k0_ref, v0_ref,
               k1_ref, v1_ref, wk_ref, wv_ref, out_ref):
  r = h1_ref.shape[0]
  m0 = _col0(r)
  parts = aggp_ref[...]
  agg = parts[0] + parts[1]
  upd1 = agg[:, 0:1] > 0.5
  out1 = jnp.where(upd1, _lnormalize(agg, m0), h1_ref[...])

  upd2 = mf_ref[...][:, 0:1] > 0.5

  q = q_ref[...]
  k0 = k0_ref[...]
  k1 = k1_ref[...]
  k2 = _proj_rows(_linT(out1, wk_ref[...]), m0)
  v0 = v0_ref[...]
  v1 = v1_ref[...]
  v2 = _proj_rows(_linT(out1, wv_ref[...]), m0)

  s0 = _lorentz_inner(q, k0, m0)
  s1 = jnp.where(upd2, _lorentz_inner(q, k1, m0), _NEG)
  s2 = jnp.where(upd1, _lorentz_inner(q, k2, m0), _NEG)
  m = jnp.maximum(jnp.maximum(s0, s1), s2)
  e0 = jnp.exp(s0 - m)
  e1 = jnp.exp(s1 - m)
  e2 = jnp.exp(s2 - m)
  c = (e0 * v0 + e1 * v1 + e2 * v2) / (e0 + e1 + e2)
  out_ref[...] = _lnormalize(c, m0)


_R = 1000
_G = N // _R

_rows = lambda i: (i, 0)
_rows3 = lambda i: (0, i, 0)
_whole = lambda i: (0, 0)

_bs_rows = pl.BlockSpec((_R, D), _rows)
_bs_w = pl.BlockSpec((D, D), _whole)
_bs_aggp = pl.BlockSpec((NC, _R, D), _rows3)

_tc_a = pl.pallas_call(
    _tc_a_body,
    grid=(_G,),
    in_specs=[_bs_rows, _bs_w],
    out_specs=[_bs_rows, _bs_rows],
    out_shape=(jax.ShapeDtypeStruct((N, D), jnp.float32),
               jax.ShapeDtypeStruct((N, D), jnp.float32)),
)

_tc_b = pl.pallas_call(
    _tc_b_body,
    grid=(_G,),
    in_specs=[_bs_aggp, _bs_rows, _bs_rows, _bs_w],
    out_specs=[_bs_rows, _bs_rows, _bs_rows],
    out_shape=(jax.ShapeDtypeStruct((N, D), jnp.float32),
               jax.ShapeDtypeStruct((N, D), jnp.float32),
               jax.ShapeDtypeStruct((N, D), jnp.float32)),
)

_tc_qkv = pl.pallas_call(
    _tc_qkv_body,
    grid=(_G,),
    in_specs=[_bs_rows, _bs_w, _bs_w, _bs_w],
    out_specs=[_bs_rows, _bs_rows, _bs_rows],
    out_shape=(jax.ShapeDtypeStruct((N, D), jnp.float32),) * 3,
)

_tc_kv = pl.pallas_call(
    _tc_kv_body,
    grid=(_G,),
    in_specs=[_bs_rows, _bs_w, _bs_w],
    out_specs=[_bs_rows, _bs_rows],
    out_shape=(jax.ShapeDtypeStruct((N, D), jnp.float32),) * 2,
)

_tc_c = pl.pallas_call(
    _tc_c_body,
    grid=(_G,),
    in_specs=[_bs_aggp] + [_bs_rows] * 7 + [_bs_w, _bs_w],
    out_specs=_bs_rows,
    out_shape=jax.ShapeDtypeStruct((N, D), jnp.float32),
)


def _prep_edges(edges):
  src = edges[0].astype(jnp.int32).reshape(NW, EPW)
  dst = edges[1].astype(jnp.int32).reshape(NW, EPW)
  pad = EPW_PAD - EPW
  src = jnp.pad(src, ((0, 0), (0, pad)), constant_values=0)
  dst = jnp.pad(dst, ((0, 0), (0, pad)), constant_values=DUMMY)
  return (src.reshape(NW * EPW_PAD), dst.reshape(NW, NCHUNK, CH))


@jax.jit
def kernel(x_H, edge_index_l1, edge_index_l2, Wq, Wk, Wv, W_agg):
  src2, dst2 = _prep_edges(edge_index_l2)
  src1, dst1 = _prep_edges(edge_index_l1)
  zeros = jnp.zeros((ZR, D), jnp.float32)

  sc_scatter = _get_sc_scatter()
  x0, h2 = _tc_a(x_H, W_agg)
  agg2p = sc_scatter(h2, src2, dst2, zeros).reshape(NC, NPAD, D)
  q, k0, v0 = _tc_qkv(x0, Wq, Wk, Wv)
  z2, m2f, h1 = _tc_b(agg2p, h2, x0, W_agg)
  agg1p = sc_scatter(h1, src1, dst1, zeros).reshape(NC, NPAD, D)
  k1, v1 = _tc_kv(z2, Wk, Wv)
  return _tc_c(agg1p, h1, m2f, q, k0, v0, k1, v1, Wk, Wv)


# speedup vs baseline: 1.1921x; 1.0664x over previous
import functools

import jax
import jax.numpy as jnp
from jax import lax
from jax.experimental import pallas as pl
from jax.experimental.pallas import tpu as pltpu
from jax.experimental.pallas import tpu_sc as plsc

N = 10000
D = 128
E = 160000

NC = 2
NS = 16
NW = NC * NS
EPW = E // NW
CH = 32
EPW_PAD = 5120
NCHUNK = EPW_PAD // CH
DUMMY = N
NPAD = 10240
RPS = NPAD // NS
ZR = 128

_NEG = -1e9


SBC = 8
NSB = NCHUNK // SBC


NBUF = 5
NG = NCHUNK // NBUF


def _sc_body(h_hbm, src_hbm, dst_hbm, zeros_hbm,
             agg_out,
             src_v, dst_v, r0, r1, r2, r3, r4,
             agg_sh, *sems):
  cid = lax.axis_index("c")
  sid = lax.axis_index("s")
  wid = cid * NS + sid
  bufs = (r0, r1, r2, r3, r4)
  gs = sems[:NBUF]
  ss = sems[NBUF:]

  pltpu.sync_copy(src_hbm.at[pl.ds(wid * EPW_PAD, EPW_PAD)], src_v)
  pltpu.sync_copy(dst_hbm.at[wid], dst_v)

  def _gather_start(j, buf, s):
    return pltpu.async_copy(h_hbm.at[src_v.at[pl.ds(j * CH, CH)]], buf, s)

  def _gather_wait(j, buf, s):
    pltpu.make_async_copy(h_hbm.at[src_v.at[pl.ds(j * CH, CH)]], buf, s).wait()

  def _scat_start(j, buf, s):
    return pltpu.async_copy(buf, agg_sh.at[dst_v.at[j]], s, add=True)

  def _scat_wait(j, buf, s):
    pltpu.make_async_copy(buf, agg_sh.at[dst_v.at[j]], s).wait()

  for b in range(NBUF):
    _gather_start(b, bufs[b], gs[b])

  base = sid * RPS
  for b in range(RPS // ZR):
    pltpu.sync_copy(zeros_hbm, agg_sh.at[pl.ds(base + b * ZR, ZR)])

  plsc.subcore_barrier()

  @pl.loop(0, NG)
  def _(g):
    for b in range(NBUF):
      j = NBUF * g + b
      _gather_wait(j, bufs[b], gs[b])
      _scat_start(j, bufs[b], ss[b])

      @pl.when(g + 1 < NG)
      def _():
        _scat_wait(j, bufs[b], ss[b])
        _gather_start(j + NBUF, bufs[b], gs[b])

  for b in range(NBUF):
    _scat_wait(NCHUNK - NBUF + b, bufs[b], ss[b])

  plsc.subcore_barrier()

  for b in range(RPS // ZR):
    off = cid * NPAD + base + b * ZR
    pltpu.sync_copy(agg_sh.at[pl.ds(base + b * ZR, ZR)],
                    agg_out.at[pl.ds(off, ZR)])


@functools.cache
def _get_sc_scatter():
  return pl.kernel(
    _sc_body,
    out_type=jax.ShapeDtypeStruct((NC * NPAD, D), jnp.float32),
    mesh=plsc.VectorSubcoreMesh(core_axis_name="c", subcore_axis_name="s",
                                num_cores=NC, num_subcores=NS),
    scratch_types=[
        pltpu.VMEM((EPW_PAD,), jnp.int32),
        pltpu.VMEM((NCHUNK, CH), jnp.int32),
    ] + [pltpu.VMEM((CH, D), jnp.float32)] * NBUF + [
        pltpu.VMEM_SHARED((NPAD, D), jnp.float32),
    ] + [pltpu.SemaphoreType.DMA] * (2 * NBUF),
  )


def _col0(r):
  return lax.broadcasted_iota(jnp.int32, (r, D), 1) == 0


def _proj_rows(y, m0):
  s2 = jnp.sum(jnp.where(m0, 0.0, y * y), axis=1, keepdims=True)
  return jnp.where(m0, jnp.sqrt(1.0 + s2), y)


def _linT(x, w):
  return lax.dot_general(x, w, (((1,), (1,)), ((), ())),
                         precision=lax.Precision.HIGHEST,
                         preferred_element_type=jnp.float32)


def _lorentz_inner(a, b, m0):
  p = a * b
  return jnp.sum(jnp.where(m0, -p, p), axis=1, keepdims=True)


def _lnormalize(c, m0):
  inner = _lorentz_inner(c, c, m0)
  return c / jnp.sqrt(jnp.clip(-inner, 1e-6, None))


def _tc_a_body(x_ref, wagg_ref, x0_ref, h2_ref):
  r = x_ref.shape[0]
  m0 = _col0(r)
  x0 = _proj_rows(x_ref[...], m0)
  h2 = _proj_rows(_linT(x0, wagg_ref[...]), m0)
  x0_ref[...] = x0
  h2_ref[...] = h2


def _tc_b_body(aggp_ref, h_ref, x0_ref, wagg_ref,
               z_ref, mf_ref, h1_ref):
  r = h_ref.shape[0]
  m0 = _col0(r)
  parts = aggp_ref[...]
  agg = parts[0] + parts[1]
  upd = agg[:, 0:1] > 0.5
  nrm = _lnormalize(agg, m0)
  h = h_ref[...]
  z = jnp.where(upd, nrm, h)
  x1 = jnp.where(upd, nrm, x0_ref[...])
  z_ref[...] = z
  mf_ref[...] = jnp.where(jnp.broadcast_to(upd, (r, D)), 1.0, 0.0)
  h1_ref[...] = _proj_rows(_linT(x1, wagg_ref[...]), m0)


def _tc_qkv_body(x_ref, wq_ref, wk_ref, wv_ref, q_ref, k_ref, v_ref):
  r = x_ref.shape[0]
  m0 = _col0(r)
  x = x_ref[...]
  q_ref[...] = _proj_rows(_linT(x, wq_ref[...]), m0)
  k_ref[...] = _proj_rows(_linT(x, wk_ref[...]), m0)
  v_ref[...] = _proj_rows(_linT(x, wv_ref[...]), m0)


def _tc_kv_body(x_ref, wk_ref, wv_ref, k_ref, v_ref):
  r = x_ref.shape[0]
  m0 = _col0(r)
  x = x_ref[...]
  k_ref[...] = _proj_rows(_linT(x, wk_ref[...]), m0)
  v_ref[...] = _proj_rows(_linT(x, wv_ref[...]), m0)


def _tc_c_body(aggp_ref, h1_ref, mf_ref, q_ref, k0_ref, v0_ref,
               k1_ref, v1_ref, wk_ref, wv_ref, out_ref):
  r = h1_ref.shape[0]
  m0 = _col0(r)
  parts = aggp_ref[...]
  agg = parts[0] + parts[1]
  upd1 = agg[:, 0:1] > 0.5
  out1 = jnp.where(upd1, _lnormalize(agg, m0), h1_ref[...])

  upd2 = mf_ref[...][:, 0:1] > 0.5

  q = q_ref[...]
  k0 = k0_ref[...]
  k1 = k1_ref[...]
  k2 = _proj_rows(_linT(out1, wk_ref[...]), m0)
  v0 = v0_ref[...]
  v1 = v1_ref[...]
  v2 = _proj_rows(_linT(out1, wv_ref[...]), m0)

  s0 = _lorentz_inner(q, k0, m0)
  s1 = jnp.where(upd2, _lorentz_inner(q, k1, m0), _NEG)
  s2 = jnp.where(upd1, _lorentz_inner(q, k2, m0), _NEG)
  m = jnp.maximum(jnp.maximum(s0, s1), s2)
  e0 = jnp.exp(s0 - m)
  e1 = jnp.exp(s1 - m)
  e2 = jnp.exp(s2 - m)
  c = (e0 * v0 + e1 * v1 + e2 * v2) / (e0 + e1 + e2)
  out_ref[...] = _lnormalize(c, m0)


_R = 2000
_G = N // _R

_rows = lambda i: (i, 0)
_rows3 = lambda i: (0, i, 0)
_whole = lambda i: (0, 0)

_bs_rows = pl.BlockSpec((_R, D), _rows)
_bs_w = pl.BlockSpec((D, D), _whole)
_bs_aggp = pl.BlockSpec((NC, _R, D), _rows3)

_tc_a = pl.pallas_call(
    _tc_a_body,
    grid=(_G,),
    in_specs=[_bs_rows, _bs_w],
    out_specs=[_bs_rows, _bs_rows],
    out_shape=(jax.ShapeDtypeStruct((N, D), jnp.float32),
               jax.ShapeDtypeStruct((N, D), jnp.float32)),
)

_tc_b = pl.pallas_call(
    _tc_b_body,
    grid=(_G,),
    in_specs=[_bs_aggp, _bs_rows, _bs_rows, _bs_w],
    out_specs=[_bs_rows, _bs_rows, _bs_rows],
    out_shape=(jax.ShapeDtypeStruct((N, D), jnp.float32),
               jax.ShapeDtypeStruct((N, D), jnp.float32),
               jax.ShapeDtypeStruct((N, D), jnp.float32)),
)

_tc_qkv = pl.pallas_call(
    _tc_qkv_body,
    grid=(_G,),
    in_specs=[_bs_rows, _bs_w, _bs_w, _bs_w],
    out_specs=[_bs_rows, _bs_rows, _bs_rows],
    out_shape=(jax.ShapeDtypeStruct((N, D), jnp.float32),) * 3,
)

_tc_kv = pl.pallas_call(
    _tc_kv_body,
    grid=(_G,),
    in_specs=[_bs_rows, _bs_w, _bs_w],
    out_specs=[_bs_rows, _bs_rows],
    out_shape=(jax.ShapeDtypeStruct((N, D), jnp.float32),) * 2,
)

_tc_c = pl.pallas_call(
    _tc_c_body,
    grid=(_G,),
    in_specs=[_bs_aggp] + [_bs_rows] * 7 + [_bs_w, _bs_w],
    out_specs=_bs_rows,
    out_shape=jax.ShapeDtypeStruct((N, D), jnp.float32),
)


def _prep_edges(edges):
  src = edges[0].astype(jnp.int32).reshape(NW, EPW)
  dst = edges[1].astype(jnp.int32).reshape(NW, EPW)
  pad = EPW_PAD - EPW
  src = jnp.pad(src, ((0, 0), (0, pad)), constant_values=0)
  dst = jnp.pad(dst, ((0, 0), (0, pad)), constant_values=DUMMY)
  return (src.reshape(NW * EPW_PAD), dst.reshape(NW, NCHUNK, CH))


@jax.jit
def kernel(x_H, edge_index_l1, edge_index_l2, Wq, Wk, Wv, W_agg):
  src2, dst2 = _prep_edges(edge_index_l2)
  src1, dst1 = _prep_edges(edge_index_l1)
  zeros = jnp.zeros((ZR, D), jnp.float32)

  sc_scatter = _get_sc_scatter()
  x0, h2 = _tc_a(x_H, W_agg)
  agg2p = sc_scatter(h2, src2, dst2, zeros).reshape(NC, NPAD, D)
  q, k0, v0 = _tc_qkv(x0, Wq, Wk, Wv)
  z2, m2f, h1 = _tc_b(agg2p, h2, x0, W_agg)
  agg1p = sc_scatter(h1, src1, dst1, zeros).reshape(NC, NPAD, D)
  k1, v1 = _tc_kv(z2, Wk, Wv)
  return _tc_c(agg1p, h1, m2f, q, k0, v0, k1, v1, Wk, Wv)
